# baseline (speedup 1.0000x reference)
import functools

import jax
import jax.numpy as jnp
from jax import lax
from jax.experimental import pallas as pl
from jax.experimental.pallas import tpu as pltpu
from jax.experimental.pallas import tpu_sc as plsc

_B, _L, _V, _C = 4096, 200, 1000, 9
_VP = 1024
_CP = 16
_NW = 32
_ROWS_PER_W = _B // _NW
_IDS_PER_W = _ROWS_PER_W * _L


def _sc_accumulate(ids_flat, table):
    mesh = plsc.VectorSubcoreMesh(core_axis_name="c", subcore_axis_name="s")

    @functools.partial(
        pl.kernel,
        mesh=mesh,
        compiler_params=pltpu.CompilerParams(
            needs_layout_passes=False, use_tc_tiling_on_sc=True),
        out_type=jax.ShapeDtypeStruct((_B * 128,), jnp.float32),
        scratch_types=[
            pltpu.VMEM((_ROWS_PER_W, _L), jnp.int32),
            pltpu.VMEM((_C * _VP,), jnp.float32),
            pltpu.VMEM((_ROWS_PER_W * 128,), jnp.float32),
        ],
    )
    def sc_kernel(ids_hbm, tab_hbm, out_hbm, ids_v, tab_v, out_v):
        num_c = lax.axis_size("c")
        wid = lax.axis_index("s") * num_c + lax.axis_index("c")
        pltpu.sync_copy(ids_hbm.at[pl.ds(wid * _ROWS_PER_W, _ROWS_PER_W)], ids_v)
        pltpu.sync_copy(tab_hbm, tab_v)

        lane = lax.iota(jnp.int32, 16)
        coffs = [jnp.full((16,), c * _VP, jnp.int32) for c in range(_C)]

        def row_body(r, _):
            def j_body(jj, accs):
                vec = ids_v[r, pl.ds(jj * 16, 16)]
                return tuple(
                    accs[c] + plsc.load_gather(tab_v, [vec + coffs[c]])
                    for c in range(_C)
                )

            accs = lax.fori_loop(
                0, _L // 16, j_body,
                tuple(jnp.zeros((16,), jnp.float32) for _ in range(_C)),
            )
            vec = ids_v[r, pl.ds(_L - 16, 16)]
            tmask = lane >= 8
            accs = tuple(
                accs[c] + jnp.where(
                    tmask, plsc.load_gather(tab_v, [vec + coffs[c]]), 0.0)
                for c in range(_C)
            )
            out = jnp.zeros((16,), jnp.float32)
            for c in range(_C):
                out = jnp.where(lane == c, jnp.sum(accs[c]), out)
            out_v[pl.ds(r * 128, _CP)] = out
            return 0

        lax.fori_loop(0, _ROWS_PER_W, row_body, 0)
        pltpu.sync_copy(
            out_v, out_hbm.at[pl.ds(wid * _ROWS_PER_W * 128, _ROWS_PER_W * 128)])

    return sc_kernel(ids_flat, table)


def _tc_log_softmax(s, b_row):
    def body(s_ref, b_ref, o_ref):
        logits = s_ref[:, :_CP] + b_ref[...]
        cls = lax.broadcasted_iota(jnp.int32, (_B, _CP), 1)
        valid = cls < _C
        m = jnp.max(jnp.where(valid, logits, -1e30), axis=1, keepdims=True)
        e = jnp.where(valid, jnp.exp(logits - m), 0.0)
        lse = jnp.log(jnp.sum(e, axis=1, keepdims=True))
        o_ref[...] = (logits - m - lse)[:, :_C]

    return pl.pallas_call(
        body,
        out_shape=jax.ShapeDtypeStruct((_B, _C), jnp.float32),
    )(s, b_row)


def kernel(ids, W, b):
    table = jnp.zeros((_C, _VP), jnp.float32).at[:, :_V].set(W).reshape(-1)
    b_row = jnp.pad(b, (0, _CP - _C))[None, :]
    s = _sc_accumulate(ids, table).reshape(_B, 128)
    return _tc_log_softmax(s, b_row)


# speedup vs baseline: 1.0627x; 1.0627x over previous
import functools

import jax
import jax.numpy as jnp
from jax import lax
from jax.experimental import pallas as pl
from jax.experimental.pallas import tpu as pltpu
from jax.experimental.pallas import tpu_sc as plsc

_B, _L, _V, _C = 4096, 200, 1000, 9
_VP = 1024
_CP = 16
_NW = 32
_ROWS_PER_W = _B // _NW
_IDS_PER_W = _ROWS_PER_W * _L


def _sc_accumulate(ids_flat, table):
    mesh = plsc.VectorSubcoreMesh(core_axis_name="c", subcore_axis_name="s")

    @functools.partial(
        pl.kernel,
        mesh=mesh,
        compiler_params=pltpu.CompilerParams(
            needs_layout_passes=False, use_tc_tiling_on_sc=True),
        out_type=jax.ShapeDtypeStruct((_B * 128,), jnp.float32),
        scratch_types=[
            pltpu.VMEM((_ROWS_PER_W, _L), jnp.int32),
            pltpu.VMEM((_VP * _CP,), jnp.float32),
            pltpu.VMEM((_ROWS_PER_W * 128,), jnp.float32),
        ],
    )
    def sc_kernel(ids_hbm, tab_hbm, out_hbm, ids_v, tab_v, out_v):
        num_c = lax.axis_size("c")
        wid = lax.axis_index("s") * num_c + lax.axis_index("c")
        pltpu.sync_copy(ids_hbm.at[pl.ds(wid * _ROWS_PER_W, _ROWS_PER_W)], ids_v)
        pltpu.sync_copy(tab_hbm, tab_v)

        zeros4 = tuple(jnp.zeros((_CP,), jnp.float32) for _ in range(4))

        def pair_body(p, _):
            r0 = p * 2

            def j_body(jj, carry):
                a, b2 = carry
                va = ids_v[r0, pl.ds(jj * 16, 16)]
                vb = ids_v[r0 + 1, pl.ds(jj * 16, 16)]
                for u in range(16):
                    a = (a[1:]) + (a[0] + tab_v[pl.ds(va[u] * _CP, _CP)],)
                    b2 = (b2[1:]) + (b2[0] + tab_v[pl.ds(vb[u] * _CP, _CP)],)
                return a, b2

            a, b2 = lax.fori_loop(0, _L // 16, j_body, (zeros4, zeros4))
            va = ids_v[r0, pl.ds(_L - 16, 16)]
            vb = ids_v[r0 + 1, pl.ds(_L - 16, 16)]
            for u in range(8, 16):
                a = (a[1:]) + (a[0] + tab_v[pl.ds(va[u] * _CP, _CP)],)
                b2 = (b2[1:]) + (b2[0] + tab_v[pl.ds(vb[u] * _CP, _CP)],)
            out_v[pl.ds(r0 * 128, _CP)] = (a[0] + a[1]) + (a[2] + a[3])
            out_v[pl.ds(r0 * 128 + 128, _CP)] = (b2[0] + b2[1]) + (b2[2] + b2[3])
            return 0

        lax.fori_loop(0, _ROWS_PER_W // 2, pair_body, 0)
        pltpu.sync_copy(
            out_v, out_hbm.at[pl.ds(wid * _ROWS_PER_W * 128, _ROWS_PER_W * 128)])

    return sc_kernel(ids_flat, table)


def _tc_log_softmax(s, b_row):
    def body(s_ref, b_ref, o_ref):
        logits = s_ref[:, :_CP] + b_ref[...]
        cls = lax.broadcasted_iota(jnp.int32, (_B, _CP), 1)
        valid = cls < _C
        m = jnp.max(jnp.where(valid, logits, -1e30), axis=1, keepdims=True)
        e = jnp.where(valid, jnp.exp(logits - m), 0.0)
        lse = jnp.log(jnp.sum(e, axis=1, keepdims=True))
        o_ref[...] = (logits - m - lse)[:, :_C]

    return pl.pallas_call(
        body,
        out_shape=jax.ShapeDtypeStruct((_B, _C), jnp.float32),
    )(s, b_row)


def kernel(ids, W, b):
    table = jnp.zeros((_VP, _CP), jnp.float32).at[:_V, :_C].set(W.T).reshape(-1)
    b_row = jnp.pad(b, (0, _CP - _C))[None, :]
    s = _sc_accumulate(ids, table).reshape(_B, 128)
    return _tc_log_softmax(s, b_row)
